# Initial kernel scaffold; baseline (speedup 1.0000x reference)
import jax
import jax.numpy as jnp
from jax.experimental import pallas as pl


def kernel(x, gate_W, gate_b, bias, sW1, sb1, sW2, sb2, eW1, eb1, eW2, eb2):
    raise NotImplementedError("write your pallas kernel here")


# speedup vs baseline: 2.3547x; 2.3547x over previous
import functools

import jax
import jax.numpy as jnp
from jax.experimental import pallas as pl
from jax.experimental.pallas import tpu as pltpu

E = 8
TOPK = 2
D = 768
H = 4 * D
T = 2048
TM = 512


def _gelu_exact(h):
    return 0.5 * h * (1.0 + jax.lax.erf(h * 0.7071067811865476))


def _router_body(x_ref, gW_ref, gb_ref, bias_ref, gatew_ref):
    x = x_ref[...]
    logits = jnp.dot(x, gW_ref[...], preferred_element_type=jnp.float32)
    logits = logits + gb_ref[...] + bias_ref[...]
    s = jax.nn.sigmoid(logits)
    e_iota = jax.lax.broadcasted_iota(jnp.int32, s.shape, 1)
    m1 = jnp.max(s, axis=1, keepdims=True)
    i1 = jnp.min(jnp.where(s == m1, e_iota, E), axis=1, keepdims=True)
    s2 = jnp.where(e_iota == i1, -jnp.inf, s)
    m2 = jnp.max(s2, axis=1, keepdims=True)
    i2 = jnp.min(jnp.where(s2 == m2, e_iota, E), axis=1, keepdims=True)
    mask = (e_iota == i1) | (e_iota == i2)
    gatew_ref[...] = jnp.where(mask, s, 0.0)


def _moe_dense_body(gatew_ref, xbf_ref, eW1_ref, eb1_ref, eW2_ref, eb2_ref,
                    sW1_ref, sb1_ref, sW2_ref, sb2_ref, out_ref):
    e = pl.program_id(0)
    t = pl.program_id(1)
    is_shared = e == E

    xb = xbf_ref[...]
    w1 = jnp.where(is_shared, sW1_ref[...], eW1_ref[0])
    w2 = jnp.where(is_shared, sW2_ref[...], eW2_ref[0])
    b1 = jnp.where(is_shared, sb1_ref[...], eb1_ref[0])
    b2 = jnp.where(is_shared, sb2_ref[...], eb2_ref[0])
    h = jnp.dot(xb, w1, preferred_element_type=jnp.float32) + b1
    h = _gelu_exact(h).astype(jnp.bfloat16)
    y = jnp.dot(h, w2, preferred_element_type=jnp.float32) + b2

    gw = gatew_ref[...]
    e_iota = jax.lax.broadcasted_iota(jnp.int32, gw.shape, 1)
    scale = jnp.sum(jnp.where(e_iota == e, gw, 0.0), axis=1, keepdims=True)
    scale = jnp.where(is_shared, 1.0, scale)
    contrib = y * scale

    sl = pl.ds(t * TM, TM)

    @pl.when(e == 0)
    def _():
        out_ref[sl, :] = contrib

    @pl.when(e > 0)
    def _():
        out_ref[sl, :] = out_ref[sl, :] + contrib


def _router(x_flat, gate_W, gate_b, bias, interpret=False):
    return pl.pallas_call(
        _router_body,
        out_shape=jax.ShapeDtypeStruct((T, E), jnp.float32),
        interpret=interpret,
    )(x_flat, gate_W, gate_b.reshape(1, E), bias.reshape(1, E))


def _moe_dense(gate_w, xbf, eW1, eb1, eW2, eb2, sW1, sb1, sW2, sb2,
               interpret=False):
    grid = (E + 1, T // TM)
    emap = lambda e, t: (jnp.minimum(e, E - 1), 0, 0)
    return pl.pallas_call(
        _moe_dense_body,
        grid=grid,
        in_specs=[
            pl.BlockSpec((TM, E), lambda e, t: (t, 0)),
            pl.BlockSpec((TM, D), lambda e, t: (t, 0)),
            pl.BlockSpec((1, D, H), emap),
            pl.BlockSpec((1, 1, H), emap),
            pl.BlockSpec((1, H, D), emap),
            pl.BlockSpec((1, 1, D), emap),
            pl.BlockSpec((D, H), lambda e, t: (0, 0)),
            pl.BlockSpec((1, H), lambda e, t: (0, 0)),
            pl.BlockSpec((H, D), lambda e, t: (0, 0)),
            pl.BlockSpec((1, D), lambda e, t: (0, 0)),
        ],
        out_specs=pl.BlockSpec((T, D), lambda e, t: (0, 0)),
        out_shape=jax.ShapeDtypeStruct((T, D), jnp.float32),
        compiler_params=pltpu.CompilerParams(
            dimension_semantics=("arbitrary", "arbitrary"),
        ),
        interpret=interpret,
    )(gate_w, xbf, eW1, eb1, eW2, eb2, sW1, sb1, sW2, sb2)


def _kernel_impl(x, gate_W, gate_b, bias, sW1, sb1, sW2, sb2,
                 eW1, eb1, eW2, eb2, interpret=False):
    B, S, Dh = x.shape
    x_flat = x.reshape(-1, Dh)
    gate_w = _router(x_flat, gate_W, gate_b, bias, interpret=interpret)
    bf = jnp.bfloat16
    out = _moe_dense(
        gate_w, x_flat.astype(bf),
        eW1.astype(bf), eb1.reshape(E, 1, H), eW2.astype(bf),
        eb2.reshape(E, 1, D),
        sW1.astype(bf), sb1.reshape(1, H), sW2.astype(bf), sb2.reshape(1, D),
        interpret=interpret)
    return out.reshape(B, S, Dh)


def kernel(x, gate_W, gate_b, bias, sW1, sb1, sW2, sb2, eW1, eb1, eW2, eb2):
    return _kernel_impl(x, gate_W, gate_b, bias, sW1, sb1, sW2, sb2,
                        eW1, eb1, eW2, eb2)
